# baseline (speedup 1.0000x reference)
import functools

import jax
import jax.numpy as jnp
from jax import lax
from jax.experimental import pallas as pl
from jax.experimental.pallas import tpu as pltpu
from jax.experimental.pallas import tpu_sc as plsc

U = 256
P = 1024
N = U + P
E = 40960
D = 128
H = 256
NC = 2
NS = 16
NW = NC * NS
NN = N * N
CHUNK = NN // NS
ZCHUNK = 12800
EPW = E // NW
UPW = U // NW
PPW = P // NW

def _sc_body(uid_hbm, pid_hbm, row_hbm, col_hbm, ut_hbm, pt_hbm,
             x_hbm, cnt_hbm,
             uidx_v, urows_v, pidx_v, prows_v, row_v, col_v,
             fidx_v, ones_v, zero_v, c_sh, sem):
    cid = lax.axis_index("c")
    sid = lax.axis_index("s")
    wid = sid * NC + cid

    @pl.loop(0, 128, step=16)
    def _(i):
        ones_v[pl.ds(i, 16)] = jnp.full((16,), 1.0, jnp.float32)

    @pl.loop(0, ZCHUNK, step=16)
    def _(i):
        zero_v[pl.ds(i, 16)] = jnp.zeros((16,), jnp.float32)

    base = sid * CHUNK

    @pl.loop(0, CHUNK, step=ZCHUNK)
    def _(off):
        pltpu.sync_copy(zero_v, c_sh.at[pl.ds(base + off, ZCHUNK)])

    pltpu.sync_copy(uid_hbm.at[pl.ds(wid * UPW, UPW)], uidx_v)
    pltpu.async_copy(ut_hbm.at[uidx_v], urows_v, sem).wait()
    pltpu.sync_copy(urows_v, x_hbm.at[pl.ds(wid * UPW, UPW)])

    pltpu.sync_copy(pid_hbm.at[pl.ds(wid * PPW, PPW)], pidx_v)
    pltpu.async_copy(pt_hbm.at[pidx_v], prows_v, sem).wait()
    pltpu.sync_copy(prows_v, x_hbm.at[pl.ds(U + wid * PPW, PPW)])

    ebase = wid * EPW
    pltpu.sync_copy(row_hbm.at[pl.ds(ebase, EPW)], row_v)
    pltpu.sync_copy(col_hbm.at[pl.ds(ebase, EPW)], col_v)

    plsc.subcore_barrier()

    @pl.loop(0, EPW, step=128)
    def _(e0):
        @pl.loop(0, 128, step=16)
        def _(j):
            r = row_v[pl.ds(e0 + j, 16)]
            c = col_v[pl.ds(e0 + j, 16)]
            fidx_v[pl.ds(j, 16)] = ((((c >> 3) * (N // 128) + (r >> 7)) << 10)
                                    + ((c & 7) << 7) + (r & 127))
        pltpu.sync_copy(ones_v, c_sh.at[fidx_v], add=True)

    plsc.subcore_barrier()

    pltpu.sync_copy(c_sh.at[pl.ds(base, CHUNK)],
                    cnt_hbm.at[cid, pl.ds(base, CHUNK)])


@functools.cache
def _sc_gather_count():
    mesh = plsc.VectorSubcoreMesh(core_axis_name="c", subcore_axis_name="s")
    return pl.kernel(
        _sc_body,
        out_type=(
            jax.ShapeDtypeStruct((N, D), jnp.float32),
            jax.ShapeDtypeStruct((NC, NN), jnp.float32),
        ),
        mesh=mesh,
        scratch_types=[
            pltpu.VMEM((UPW,), jnp.int32),
            pltpu.VMEM((UPW, D), jnp.float32),
            pltpu.VMEM((PPW,), jnp.int32),
            pltpu.VMEM((PPW, D), jnp.float32),
            pltpu.VMEM((EPW,), jnp.int32),
            pltpu.VMEM((EPW,), jnp.int32),
            pltpu.VMEM((128,), jnp.int32),
            pltpu.VMEM((128,), jnp.float32),
            pltpu.VMEM((ZCHUNK,), jnp.float32),
            pltpu.VMEM_SHARED((NN,), jnp.float32),
            pltpu.SemaphoreType.DMA,
        ],
    )


def _dot(a, b):
    return jnp.dot(a, b, preferred_element_type=jnp.float32)


PB = 32
PS = 8


def _tc_body(x_ref, c_ref, w0_ref, b0_ref, w1_ref, b1_ref, w2_ref, b2_ref,
             wu_ref, bu_ref, wp_ref, bp_ref, wq1_ref, bq1_ref, wbd_ref,
             bq2_ref, out_ref, bp_s):
    c4 = (c_ref[0] + c_ref[1]).reshape(N // 8, N // 128, 8, 128)
    cs = jnp.concatenate(
        [c4[:, b].reshape(N, 128) for b in range(N // 128)], axis=1)
    ri = lax.broadcasted_iota(jnp.int32, (N, N), 0)
    ci = lax.broadcasted_iota(jnp.int32, (N, N), 1)
    cs = cs + jnp.where(ri == ci, 1.0, 0.0)
    deg = jnp.sum(cs, axis=1, keepdims=True)
    dis = lax.rsqrt(deg)
    cs_b = cs.astype(jnp.bfloat16)

    h = x_ref[...]
    for w_ref, b_ref in ((w0_ref, b0_ref), (w1_ref, b1_ref),
                         (w2_ref, b2_ref)):
        y = (dis * _dot(h, w_ref[...])).astype(jnp.bfloat16)
        h = jnp.maximum(dis * _dot(cs_b, y) + b_ref[...], 0.0)

    uf = _dot(h[:U], wu_ref[...]) + bu_ref[...]
    pf = _dot(h[U:], wp_ref[...]) + bp_ref[...]
    au = (_dot(uf, wq1_ref[:D]) + bq1_ref[...]).astype(jnp.bfloat16)
    bp_s[...] = _dot(pf, wq1_ref[D:]).astype(jnp.bfloat16)
    wbd = wbd_ref[...]
    bq2 = bq2_ref[0, 0]
    zero_b = jnp.bfloat16(0)

    def pair_block(i, carry):
        rows = []
        for s in range(PS):
            pieces = []
            for j16 in range(0, PB, 16):
                slab = bp_s[pl.ds(i * (PS * PB) + s * PB + j16, 16), :]
                for j in range(16):
                    pieces.append(jnp.maximum(au + slab[j:j + 1, :], zero_b))
            rows.append(jnp.concatenate(pieces, axis=1))
        t = jnp.concatenate(rows, axis=0)
        z = _dot(t, wbd)
        zst = jnp.concatenate(
            [z[s * U:(s + 1) * U, :] for s in range(PS)], axis=1)
        out_ref[:, pl.ds(i * (PS * PB), PS * PB)] = jax.nn.sigmoid(zst + bq2)
        return carry

    lax.fori_loop(0, P // (PS * PB), pair_block, 0)


_tc_dense = pl.pallas_call(
    _tc_body,
    out_shape=jax.ShapeDtypeStruct((U, P), jnp.float32),
    scratch_shapes=[pltpu.VMEM((P, H), jnp.bfloat16)],
)


def kernel(user_ids, paper_ids, edge_index, user_paper_interactions,
           user_table, paper_table, W0, b0, W1, b1, W2, b2,
           Wu, bu, Wp, bp, Wq1, bq1, Wq2, bq2):
    del user_paper_interactions
    uid = user_ids.astype(jnp.int32)
    pid = paper_ids.astype(jnp.int32)
    row = edge_index[0].astype(jnp.int32)
    col = edge_index[1].astype(jnp.int32)
    x, cnt = _sc_gather_count()(uid, pid, row, col, user_table, paper_table)
    cnt = cnt.reshape(NC, NN // 128, 128)
    blk = jnp.arange(PB * H, dtype=jnp.int32) // H
    wrep = jnp.tile(Wq2[:, 0].astype(jnp.bfloat16), PB)
    wbd = jnp.where(blk[:, None] == jnp.arange(PB, dtype=jnp.int32)[None, :],
                    wrep[:, None], jnp.bfloat16(0))
    return _tc_dense(
        x, cnt,
        W0, b0.reshape(1, H), W1, b1.reshape(1, H), W2, b2.reshape(1, H),
        Wu, bu.reshape(1, D), Wp, bp.reshape(1, D),
        Wq1, bq1.reshape(1, H), wbd, bq2.reshape(1, 1))


# speedup vs baseline: 1.2693x; 1.2693x over previous
import functools

import jax
import jax.numpy as jnp
from jax import lax
from jax.experimental import pallas as pl
from jax.experimental.pallas import tpu as pltpu
from jax.experimental.pallas import tpu_sc as plsc

U = 256
P = 1024
N = U + P
E = 40960
D = 128
H = 256
NC = 2
NS = 16
NW = NC * NS
NN = N * N
HALF = NN // NC
CHUNK = HALF // NS
ZCHUNK = 12800
EPC = E // NS
UPW = U // NW
PPW = P // NW

def _sc_body(uid_hbm, pid_hbm, row_hbm, col_hbm, ut_hbm, pt_hbm,
             x_hbm, cnt_hbm,
             uidx_v, urows_v, pidx_v, prows_v, row_v, col_v,
             fidx_v, ones_v, zero_v, c_sh, sem):
    cid = lax.axis_index("c")
    sid = lax.axis_index("s")
    wid = sid * NC + cid

    @pl.loop(0, 128, step=16)
    def _(i):
        ones_v[pl.ds(i, 16)] = jnp.full((16,), 1.0, jnp.float32)

    @pl.loop(0, ZCHUNK, step=16)
    def _(i):
        zero_v[pl.ds(i, 16)] = jnp.zeros((16,), jnp.float32)

    base = sid * CHUNK

    @pl.loop(0, CHUNK, step=ZCHUNK)
    def _(off):
        pltpu.sync_copy(zero_v, c_sh.at[pl.ds(base + off, ZCHUNK)])

    pltpu.sync_copy(uid_hbm.at[pl.ds(wid * UPW, UPW)], uidx_v)
    pltpu.async_copy(ut_hbm.at[uidx_v], urows_v, sem).wait()
    pltpu.sync_copy(urows_v, x_hbm.at[pl.ds(wid * UPW, UPW)])

    pltpu.sync_copy(pid_hbm.at[pl.ds(wid * PPW, PPW)], pidx_v)
    pltpu.async_copy(pt_hbm.at[pidx_v], prows_v, sem).wait()
    pltpu.sync_copy(prows_v, x_hbm.at[pl.ds(U + wid * PPW, PPW)])

    ebase = sid * EPC
    pltpu.sync_copy(row_hbm.at[pl.ds(ebase, EPC)], row_v)
    pltpu.sync_copy(col_hbm.at[pl.ds(ebase, EPC)], col_v)

    plsc.subcore_barrier()

    dbase = cid * (N // NC)
    dump = HALF + sid * 16 + lax.iota(jnp.int32, 16)

    @pl.loop(0, EPC, step=128)
    def _(e0):
        @pl.loop(0, 128, step=16)
        def _(j):
            r = row_v[pl.ds(e0 + j, 16)]
            c = col_v[pl.ds(e0 + j, 16)]
            ud = c - dbase
            tiled = ((((ud >> 3) * (N // 128) + (r >> 7)) << 10)
                     + ((ud & 7) << 7) + (r & 127))
            ok = (ud >= 0) & (ud < N // NC)
            fidx_v[pl.ds(j, 16)] = jnp.where(ok, tiled, dump)
        pltpu.sync_copy(ones_v, c_sh.at[fidx_v], add=True)

    plsc.subcore_barrier()

    pltpu.sync_copy(c_sh.at[pl.ds(base, CHUNK)],
                    cnt_hbm.at[pl.ds(cid * HALF + base, CHUNK)])


@functools.cache
def _sc_gather_count():
    mesh = plsc.VectorSubcoreMesh(core_axis_name="c", subcore_axis_name="s")
    return pl.kernel(
        _sc_body,
        out_type=(
            jax.ShapeDtypeStruct((N, D), jnp.float32),
            jax.ShapeDtypeStruct((NN,), jnp.float32),
        ),
        mesh=mesh,
        scratch_types=[
            pltpu.VMEM((UPW,), jnp.int32),
            pltpu.VMEM((UPW, D), jnp.float32),
            pltpu.VMEM((PPW,), jnp.int32),
            pltpu.VMEM((PPW, D), jnp.float32),
            pltpu.VMEM((EPC,), jnp.int32),
            pltpu.VMEM((EPC,), jnp.int32),
            pltpu.VMEM((128,), jnp.int32),
            pltpu.VMEM((128,), jnp.float32),
            pltpu.VMEM((ZCHUNK,), jnp.float32),
            pltpu.VMEM_SHARED((HALF + NS * 16,), jnp.float32),
            pltpu.SemaphoreType.DMA,
        ],
    )


def _dot(a, b):
    return jnp.dot(a, b, preferred_element_type=jnp.float32)


PB = 32
PS = 8


def _tc_body(x_ref, c_ref, w0_ref, b0_ref, w1_ref, b1_ref, w2_ref, b2_ref,
             wu_ref, bu_ref, wp_ref, bp_ref, wq1_ref, bq1_ref, wbd_ref,
             bq2_ref, out_ref, bp_s):
    c4 = c_ref[...].reshape(N // 8, N // 128, 8, 128)
    cs = jnp.concatenate(
        [c4[:, b].reshape(N, 128) for b in range(N // 128)], axis=1)
    ri = lax.broadcasted_iota(jnp.int32, (N, N), 0)
    ci = lax.broadcasted_iota(jnp.int32, (N, N), 1)
    cs = cs + jnp.where(ri == ci, 1.0, 0.0)
    deg = jnp.sum(cs, axis=1, keepdims=True)
    dis = lax.rsqrt(deg)
    cs_b = cs.astype(jnp.bfloat16)

    h = x_ref[...]
    for w_ref, b_ref in ((w0_ref, b0_ref), (w1_ref, b1_ref),
                         (w2_ref, b2_ref)):
        y = (dis * _dot(h, w_ref[...])).astype(jnp.bfloat16)
        h = jnp.maximum(dis * _dot(cs_b, y) + b_ref[...], 0.0)

    uf = _dot(h[:U], wu_ref[...]) + bu_ref[...]
    pf = _dot(h[U:], wp_ref[...]) + bp_ref[...]
    au = (_dot(uf, wq1_ref[:D]) + bq1_ref[...]).astype(jnp.bfloat16)
    bp_s[...] = _dot(pf, wq1_ref[D:]).astype(jnp.bfloat16)
    wbd = wbd_ref[...]
    bq2 = bq2_ref[0, 0]
    zero_b = jnp.bfloat16(0)

    def pair_block(i, carry):
        rows = []
        for s in range(PS):
            pieces = []
            for j16 in range(0, PB, 16):
                slab = bp_s[pl.ds(i * (PS * PB) + s * PB + j16, 16), :]
                for j in range(16):
                    pieces.append(jnp.maximum(au + slab[j:j + 1, :], zero_b))
            rows.append(jnp.concatenate(pieces, axis=1))
        t = jnp.concatenate(rows, axis=0)
        z = _dot(t, wbd)
        zst = jnp.concatenate(
            [z[s * U:(s + 1) * U, :] for s in range(PS)], axis=1)
        out_ref[:, pl.ds(i * (PS * PB), PS * PB)] = jax.nn.sigmoid(zst + bq2)
        return carry

    lax.fori_loop(0, P // (PS * PB), pair_block, 0)


_tc_dense = pl.pallas_call(
    _tc_body,
    out_shape=jax.ShapeDtypeStruct((U, P), jnp.float32),
    scratch_shapes=[pltpu.VMEM((P, H), jnp.bfloat16)],
)


def kernel(user_ids, paper_ids, edge_index, user_paper_interactions,
           user_table, paper_table, W0, b0, W1, b1, W2, b2,
           Wu, bu, Wp, bp, Wq1, bq1, Wq2, bq2):
    del user_paper_interactions
    uid = user_ids.astype(jnp.int32)
    pid = paper_ids.astype(jnp.int32)
    row = edge_index[0].astype(jnp.int32)
    col = edge_index[1].astype(jnp.int32)
    x, cnt = _sc_gather_count()(uid, pid, row, col, user_table, paper_table)
    cnt = cnt.reshape(NN // 128, 128)
    blk = jnp.arange(PB * H, dtype=jnp.int32) // H
    wrep = jnp.tile(Wq2[:, 0].astype(jnp.bfloat16), PB)
    wbd = jnp.where(blk[:, None] == jnp.arange(PB, dtype=jnp.int32)[None, :],
                    wrep[:, None], jnp.bfloat16(0))
    return _tc_dense(
        x, cnt,
        W0, b0.reshape(1, H), W1, b1.reshape(1, H), W2, b2.reshape(1, H),
        Wu, bu.reshape(1, D), Wp, bp.reshape(1, D),
        Wq1, bq1.reshape(1, H), wbd, bq2.reshape(1, 1))
